# Initial kernel scaffold; baseline (speedup 1.0000x reference)
import jax
import jax.numpy as jnp
from jax.experimental import pallas as pl


def kernel(edge_index, edge_weight, home, away, embed, W_rel1, b_rel1, W_root1, W_rel2, b_rel2, W_root2, W_rel3, b_rel3, W_root3, Wd1, bd1, Wd2, bd2, Wd3, bd3, Wd4, bd4, Wd5, bd5):
    raise NotImplementedError("write your pallas kernel here")


# speedup vs baseline: 5.9014x; 5.9014x over previous
import functools

import jax
import jax.numpy as jnp
from jax import lax
from jax.experimental import pallas as pl
from jax.experimental.pallas import tpu as pltpu
from jax.experimental.pallas import tpu_sc as plsc

N_NODES = 100000
E_EDGES = 1600000
B_PAIRS = 16384
D = 32

HALF = N_NODES // 2
ROWS_PER_TILE = HALF // 16
ACC_ROWS = 51200
ZERO_STRIPE = ACC_ROWS // 16

CHUNK = 512
NCHUNK = -(-E_EDGES // (16 * CHUNK))
E_PAD = NCHUNK * 16 * CHUNK
ROWS128_PER_TILE = (NCHUNK * CHUNK) // 128


def _leaky(x):
    return jnp.where(x >= 0, x, 0.01 * x)


_MESH = plsc.VectorSubcoreMesh(core_axis_name="c", subcore_axis_name="s")
_SC_PARAMS = pltpu.CompilerParams(use_tc_tiling_on_sc=False)
_BCAST_DN = lax.GatherDimensionNumbers(
    offset_dims=(), collapsed_slice_dims=(0,), start_index_map=(0,))


def _lane_bcast(vec16, j):
    return lax.gather(vec16, jnp.full((16, 1), j, jnp.int32), _BCAST_DN, (1,),
                      mode=lax.GatherScatterMode.PROMISE_IN_BOUNDS)


@functools.partial(
    pl.kernel,
    out_type=jax.ShapeDtypeStruct((2 * ACC_ROWS, D), jnp.float32),
    mesh=_MESH,
    compiler_params=_SC_PARAMS,
    scratch_types=[
        pltpu.VMEM((4, 128), jnp.int32),
        pltpu.VMEM((4, 128), jnp.int32),
        pltpu.VMEM((4, 128), jnp.float32),
        pltpu.VMEM((CHUNK, D), jnp.float32),
        pltpu.VMEM((1, 128), jnp.int32),
        pltpu.VMEM_SHARED((ACC_ROWS, D), jnp.float32),
        pltpu.SemaphoreType.DMA,
    ],
)
def _segsum(src_hbm, dst_hbm, w_hbm, x_hbm, out_hbm,
            src_v, dst_v, w_v, rows_v, idx_v, acc, sem):
    c = lax.axis_index("c")
    s = lax.axis_index("s")
    lo = c * HALF
    iota = lax.iota(jnp.int32, 16)
    z16 = jnp.zeros((16,), jnp.float32)

    def zrow(r, carry):
        rows_v[r, pl.ds(0, 16)] = z16
        rows_v[r, pl.ds(16, 16)] = z16
        return carry

    lax.fori_loop(0, CHUNK, zrow, 0)

    def zdma(zi, carry):
        pltpu.sync_copy(rows_v, acc.at[pl.ds(s * ZERO_STRIPE + zi * CHUNK, CHUNK)])
        return carry

    lax.fori_loop(0, ZERO_STRIPE // CHUNK, zdma, 0)
    pltpu.sync_copy(
        rows_v.at[pl.ds(0, ZERO_STRIPE % CHUNK)],
        acc.at[pl.ds(s * ZERO_STRIPE + (ZERO_STRIPE // CHUNK) * CHUNK,
                     ZERO_STRIPE % CHUNK)])
    plsc.subcore_barrier()

    def chunk(ci, carry):
        row0 = s * ROWS128_PER_TILE + ci * 4
        pltpu.sync_copy(src_hbm.at[pl.ds(row0, 4)], src_v)
        pltpu.sync_copy(dst_hbm.at[pl.ds(row0, 4)], dst_v)
        pltpu.sync_copy(w_hbm.at[pl.ds(row0, 4)], w_v)
        copies = [
            pltpu.async_copy(
                x_hbm.at[src_v.at[g]], rows_v.at[pl.ds(g * 128, 128)], sem)
            for g in range(4)
        ]
        for cp in copies:
            cp.wait()

        def group(g, inner):
            for k in range(8):
                dst16 = dst_v[g, pl.ds(k * 16, 16)]
                w16 = w_v[g, pl.ds(k * 16, 16)]
                inr = (dst16 >= lo) & (dst16 < lo + HALF)
                idx16 = jnp.where(inr, dst16 - lo, HALF + s * 16 + iota)
                idx_v[0, pl.ds(k * 16, 16)] = idx16
                for j in range(16):
                    e = g * 128 + k * 16 + j
                    b = _lane_bcast(w16, j)
                    rows_v[e, pl.ds(0, 16)] = rows_v[e, pl.ds(0, 16)] * b
                    rows_v[e, pl.ds(16, 16)] = rows_v[e, pl.ds(16, 16)] * b
            pltpu.sync_copy(rows_v.at[pl.ds(g * 128, 128)],
                            acc.at[idx_v.at[0]], add=True)
            return inner

        lax.fori_loop(0, CHUNK // 128, group, 0)
        return carry

    lax.fori_loop(0, NCHUNK, chunk, 0)
    plsc.subcore_barrier()
    pltpu.sync_copy(
        acc.at[pl.ds(s * ZERO_STRIPE, ZERO_STRIPE)],
        out_hbm.at[pl.ds(c * ACC_ROWS + s * ZERO_STRIPE, ZERO_STRIPE)])


@functools.partial(
    pl.kernel,
    out_type=(jax.ShapeDtypeStruct((B_PAIRS, D), jnp.float32),
              jax.ShapeDtypeStruct((B_PAIRS, D), jnp.float32)),
    mesh=_MESH,
    compiler_params=_SC_PARAMS,
    scratch_types=[
        pltpu.VMEM((8, 128), jnp.int32),
        pltpu.VMEM((8, 128), jnp.int32),
        pltpu.VMEM((512, D), jnp.float32),
        pltpu.VMEM((512, D), jnp.float32),
        pltpu.SemaphoreType.DMA,
    ],
)
def _gather_pair(x_hbm, home_hbm, away_hbm, oh_hbm, oa_hbm,
                 hidx_v, aidx_v, bufh, bufa, sem):
    c = lax.axis_index("c")
    s = lax.axis_index("s")
    wid = s * 2 + c
    base = wid * 512
    base8 = (wid // 2) * 8
    sub = (wid % 2) * 4
    pltpu.sync_copy(home_hbm.at[pl.ds(base8, 8)], hidx_v)
    pltpu.sync_copy(away_hbm.at[pl.ds(base8, 8)], aidx_v)
    copies = []
    for q in range(4):
        copies.append(pltpu.async_copy(
            x_hbm.at[hidx_v.at[sub + q]], bufh.at[pl.ds(q * 128, 128)], sem))
        copies.append(pltpu.async_copy(
            x_hbm.at[aidx_v.at[sub + q]], bufa.at[pl.ds(q * 128, 128)], sem))
    for cp in copies:
        cp.wait()
    pltpu.sync_copy(bufh, oh_hbm.at[pl.ds(base, 512)])
    pltpu.sync_copy(bufa, oa_hbm.at[pl.ds(base, 512)])


_DENSE_BLK = 2000


def _dense_body(agg_ref, x_ref, wr_ref, br_ref, wo_ref, o_ref):
    agg = agg_ref[...]
    x = x_ref[...]
    y = lax.dot_general(agg, wr_ref[...], (((1,), (1,)), ((), ())),
                        precision=lax.Precision.HIGHEST,
                        preferred_element_type=jnp.float32)
    y = y + br_ref[...][None, :]
    y = y + lax.dot_general(x, wo_ref[...], (((1,), (1,)), ((), ())),
                            precision=lax.Precision.HIGHEST,
                            preferred_element_type=jnp.float32)
    o_ref[...] = _leaky(y)


def _dense(agg, x, Wr, br, Wo):
    grid = N_NODES // _DENSE_BLK
    return pl.pallas_call(
        _dense_body,
        grid=(grid,),
        in_specs=[
            pl.BlockSpec((_DENSE_BLK, D), lambda i: (i, 0)),
            pl.BlockSpec((_DENSE_BLK, D), lambda i: (i, 0)),
            pl.BlockSpec((D, D), lambda i: (0, 0)),
            pl.BlockSpec((D,), lambda i: (0,)),
            pl.BlockSpec((D, D), lambda i: (0, 0)),
        ],
        out_specs=pl.BlockSpec((_DENSE_BLK, D), lambda i: (i, 0)),
        out_shape=jax.ShapeDtypeStruct((N_NODES, D), jnp.float32),
    )(agg, x, Wr, br, Wo)


_HEAD_BLK = 2048


def _head_body(oh_ref, oa_ref, w1h_ref, w1a_ref, b1_ref,
               w2_ref, b2_ref, w3_ref, b3_ref, w4_ref, b4_ref,
               w5_ref, b5_ref, o_ref):
    def mm(a, w):
        return lax.dot_general(a, w, (((1,), (1,)), ((), ())),
                               precision=lax.Precision.HIGHEST,
                               preferred_element_type=jnp.float32)

    h = mm(oh_ref[...], w1h_ref[...]) + mm(oa_ref[...], w1a_ref[...])
    h = _leaky(h + b1_ref[...][None, :])
    for w_ref, b_ref in ((w2_ref, b2_ref), (w3_ref, b3_ref), (w4_ref, b4_ref)):
        h = _leaky(mm(h, w_ref[...]) + b_ref[...][None, :])
    h = _leaky(mm(h, w5_ref[...]) + b5_ref[...][None, :])
    m = jnp.max(h, axis=1, keepdims=True)
    sh = h - m
    lse = jnp.log(jnp.sum(jnp.exp(sh), axis=1, keepdims=True))
    o_ref[...] = sh - lse


def _head(oh, oa, w1h, w1a, b1, w2, b2, w3, b3, w4, b4, w5, b5):
    grid = B_PAIRS // _HEAD_BLK
    full = lambda shape: pl.BlockSpec(shape, lambda i: tuple(0 for _ in shape))
    return pl.pallas_call(
        _head_body,
        grid=(grid,),
        in_specs=[
            pl.BlockSpec((_HEAD_BLK, D), lambda i: (i, 0)),
            pl.BlockSpec((_HEAD_BLK, D), lambda i: (i, 0)),
            full((8, D)), full((8, D)), full((8,)),
            full((8, 8)), full((8,)),
            full((8, 8)), full((8,)),
            full((8, 8)), full((8,)),
            full((3, 8)), full((3,)),
        ],
        out_specs=pl.BlockSpec((_HEAD_BLK, 3), lambda i: (i, 0)),
        out_shape=jax.ShapeDtypeStruct((B_PAIRS, 3), jnp.float32),
    )(oh, oa, w1h, w1a, b1, w2, b2, w3, b3, w4, b4, w5, b5)


def kernel(edge_index, edge_weight, home, away, embed,
           W_rel1, b_rel1, W_root1, W_rel2, b_rel2, W_root2,
           W_rel3, b_rel3, W_root3,
           Wd1, bd1, Wd2, bd2, Wd3, bd3, Wd4, bd4, Wd5, bd5):
    pad = E_PAD - E_EDGES
    src2 = jnp.pad(edge_index[0], (0, pad)).reshape(-1, 128)
    dst2 = jnp.pad(edge_index[1], (0, pad)).reshape(-1, 128)
    w2 = jnp.pad(edge_weight, (0, pad)).reshape(-1, 128)
    home2 = home.reshape(-1, 128)
    away2 = away.reshape(-1, 128)

    x = embed
    for Wr, br, Wo in ((W_rel1, b_rel1, W_root1),
                       (W_rel2, b_rel2, W_root2),
                       (W_rel3, b_rel3, W_root3)):
        aggfull = _segsum(src2, dst2, w2, x)
        agg = jnp.concatenate(
            [aggfull[:HALF], aggfull[ACC_ROWS:ACC_ROWS + HALF]], axis=0)
        x = _dense(agg, x, Wr, br, Wo)

    oh, oa = _gather_pair(x, home2, away2)
    return _head(oh, oa, Wd1[:, :D], Wd1[:, D:], bd1,
                 Wd2, bd2, Wd3, bd3, Wd4, bd4, Wd5, bd5)


# speedup vs baseline: 6.8683x; 1.1638x over previous
import functools

import jax
import jax.numpy as jnp
from jax import lax
from jax.experimental import pallas as pl
from jax.experimental.pallas import tpu as pltpu
from jax.experimental.pallas import tpu_sc as plsc

N_NODES = 100000
NP = 100096
E_EDGES = 1600000
B_PAIRS = 16384
D = 32
DH = 16

CHUNK = 512
NCHUNK = -(-E_EDGES // (16 * CHUNK))
E_PAD = NCHUNK * 16 * CHUNK
ROWS128_PER_TILE = (NCHUNK * CHUNK) // 128
ZSTRIPE = NP // 16


def _leaky(x):
    return jnp.where(x >= 0, x, 0.01 * x)


_MESH = plsc.VectorSubcoreMesh(core_axis_name="c", subcore_axis_name="s")
_SC_PARAMS = pltpu.CompilerParams(use_tc_tiling_on_sc=False)
_BCAST_DN = lax.GatherDimensionNumbers(
    offset_dims=(), collapsed_slice_dims=(0,), start_index_map=(0,))


def _lane_bcast(vec16, j):
    return lax.gather(vec16, jnp.full((16, 1), j, jnp.int32), _BCAST_DN, (1,),
                      mode=lax.GatherScatterMode.PROMISE_IN_BOUNDS)


@functools.partial(
    pl.kernel,
    out_type=(jax.ShapeDtypeStruct((NP, DH), jnp.float32),
              jax.ShapeDtypeStruct((NP, DH), jnp.float32)),
    mesh=_MESH,
    compiler_params=_SC_PARAMS,
    scratch_types=[
        pltpu.VMEM((4, 128), jnp.int32),
        pltpu.VMEM((4, 128), jnp.int32),
        pltpu.VMEM((4, 128), jnp.float32),
        pltpu.VMEM((4, 128), jnp.int32),
        pltpu.VMEM((CHUNK, DH), jnp.float32),
        pltpu.VMEM((4, 128), jnp.int32),
        pltpu.VMEM((4, 128), jnp.int32),
        pltpu.VMEM((4, 128), jnp.float32),
        pltpu.VMEM((4, 128), jnp.int32),
        pltpu.VMEM((CHUNK, DH), jnp.float32),
        pltpu.VMEM_SHARED((NP, DH), jnp.float32),
        pltpu.SemaphoreType.DMA,
        pltpu.SemaphoreType.DMA,
        pltpu.SemaphoreType.DMA,
    ],
)
def _segsum(src_hbm, dst_hbm, w_hbm, x2_hbm, outl_hbm, outr_hbm,
            src_a, dst_a, w_a, gidx_a, rows_a,
            src_b, dst_b, w_b, gidx_b, rows_b,
            acc, sem_g, sem_sa, sem_sb):
    c = lax.axis_index("c")
    s = lax.axis_index("s")
    z16 = jnp.zeros((16,), jnp.float32)

    def zrow(r, carry):
        rows_a[r, pl.ds(0, 16)] = z16
        return carry

    lax.fori_loop(0, CHUNK, zrow, 0)

    def zdma(zi, carry):
        pltpu.sync_copy(rows_a, acc.at[pl.ds(s * ZSTRIPE + zi * CHUNK, CHUNK)])
        return carry

    lax.fori_loop(0, ZSTRIPE // CHUNK, zdma, 0)
    pltpu.sync_copy(
        rows_a.at[pl.ds(0, ZSTRIPE % CHUNK)],
        acc.at[pl.ds(s * ZSTRIPE + (ZSTRIPE // CHUNK) * CHUNK,
                     ZSTRIPE % CHUNK)])
    plsc.subcore_barrier()

    def do_super(ci, not_first, src_v, dst_v, w_v, gidx, rows, sem_s):
        @pl.when(not_first)
        def _():
            for g in range(4):
                pltpu.make_async_copy(
                    rows.at[pl.ds(g * 128, 128)],
                    acc.at[dst_v.at[g]], sem_s).wait()

        row0 = s * ROWS128_PER_TILE + ci * 4
        pltpu.sync_copy(src_hbm.at[pl.ds(row0, 4)], src_v)
        pltpu.sync_copy(dst_hbm.at[pl.ds(row0, 4)], dst_v)
        pltpu.sync_copy(w_hbm.at[pl.ds(row0, 4)], w_v)
        coff = c * N_NODES
        for g in range(4):
            for k in range(8):
                gidx[g, pl.ds(k * 16, 16)] = (
                    src_v[g, pl.ds(k * 16, 16)] + coff)
        for g in range(4):
            pltpu.async_copy(x2_hbm.at[gidx.at[g]],
                             rows.at[pl.ds(g * 128, 128)], sem_g)
        for g in range(4):
            pltpu.make_async_copy(x2_hbm.at[gidx.at[g]],
                                  rows.at[pl.ds(g * 128, 128)], sem_g).wait()

        def mul_group(g, carry):
            for k in range(8):
                w16 = w_v[g, pl.ds(k * 16, 16)]
                for j in range(16):
                    e = g * 128 + k * 16 + j
                    b = _lane_bcast(w16, j)
                    rows[e, pl.ds(0, 16)] = rows[e, pl.ds(0, 16)] * b
            return carry

        lax.fori_loop(0, 4, mul_group, 0)
        for g in range(4):
            pltpu.async_copy(rows.at[pl.ds(g * 128, 128)],
                             acc.at[dst_v.at[g]], sem_s, add=True)

    def pair(pi, carry):
        do_super(2 * pi, pi > 0, src_a, dst_a, w_a, gidx_a, rows_a, sem_sa)
        do_super(2 * pi + 1, pi > 0, src_b, dst_b, w_b, gidx_b, rows_b, sem_sb)
        return carry

    lax.fori_loop(0, NCHUNK // 2, pair, 0)
    for g in range(4):
        pltpu.make_async_copy(rows_a.at[pl.ds(g * 128, 128)],
                              acc.at[dst_a.at[g]], sem_sa).wait()
        pltpu.make_async_copy(rows_b.at[pl.ds(g * 128, 128)],
                              acc.at[dst_b.at[g]], sem_sb).wait()
    plsc.subcore_barrier()

    @pl.when(c == 0)
    def _():
        pltpu.sync_copy(acc.at[pl.ds(s * ZSTRIPE, ZSTRIPE)],
                        outl_hbm.at[pl.ds(s * ZSTRIPE, ZSTRIPE)])

    @pl.when(c == 1)
    def _():
        pltpu.sync_copy(acc.at[pl.ds(s * ZSTRIPE, ZSTRIPE)],
                        outr_hbm.at[pl.ds(s * ZSTRIPE, ZSTRIPE)])


@functools.partial(
    pl.kernel,
    out_type=(jax.ShapeDtypeStruct((B_PAIRS, D), jnp.float32),
              jax.ShapeDtypeStruct((B_PAIRS, D), jnp.float32)),
    mesh=_MESH,
    compiler_params=_SC_PARAMS,
    scratch_types=[
        pltpu.VMEM((8, 128), jnp.int32),
        pltpu.VMEM((8, 128), jnp.int32),
        pltpu.VMEM((512, D), jnp.float32),
        pltpu.VMEM((512, D), jnp.float32),
        pltpu.SemaphoreType.DMA,
    ],
)
def _gather_pair(x_hbm, home_hbm, away_hbm, oh_hbm, oa_hbm,
                 hidx_v, aidx_v, bufh, bufa, sem):
    c = lax.axis_index("c")
    s = lax.axis_index("s")
    wid = s * 2 + c
    base = wid * 512
    base8 = (wid // 2) * 8
    sub = (wid % 2) * 4
    pltpu.sync_copy(home_hbm.at[pl.ds(base8, 8)], hidx_v)
    pltpu.sync_copy(away_hbm.at[pl.ds(base8, 8)], aidx_v)
    copies = []
    for q in range(4):
        copies.append(pltpu.async_copy(
            x_hbm.at[hidx_v.at[sub + q]], bufh.at[pl.ds(q * 128, 128)], sem))
        copies.append(pltpu.async_copy(
            x_hbm.at[aidx_v.at[sub + q]], bufa.at[pl.ds(q * 128, 128)], sem))
    for cp in copies:
        cp.wait()
    pltpu.sync_copy(bufh, oh_hbm.at[pl.ds(base, 512)])
    pltpu.sync_copy(bufa, oa_hbm.at[pl.ds(base, 512)])


_DENSE_BLK = 2000


_NB = N_NODES // _DENSE_BLK


def _dense_body(split_out, aggl_ref, aggr_ref, x2l_ref, x2h_ref,
                wr_ref, br_ref, wo_ref, o_ref):
    x = jnp.concatenate([x2l_ref[...], x2h_ref[...]], axis=1)
    agg = jnp.concatenate([aggl_ref[...], aggr_ref[...]], axis=1)
    y = lax.dot_general(agg, wr_ref[...], (((1,), (1,)), ((), ())),
                        precision=lax.Precision.HIGHEST,
                        preferred_element_type=jnp.float32)
    y = y + br_ref[...][None, :]
    y = y + lax.dot_general(x, wo_ref[...], (((1,), (1,)), ((), ())),
                            precision=lax.Precision.HIGHEST,
                            preferred_element_type=jnp.float32)
    y = _leaky(y)
    if split_out:
        lower = pl.program_id(0) < _NB
        o_ref[...] = jnp.where(lower, y[:, :DH], y[:, DH:])
    else:
        o_ref[...] = y


def _dense(aggl, aggr, x2, Wr, br, Wo, split_out):
    nb = _NB
    if split_out:
        grid = 2 * nb
        out_shape = jax.ShapeDtypeStruct((2 * N_NODES, DH), jnp.float32)
        out_spec = pl.BlockSpec((_DENSE_BLK, DH), lambda i: (i, 0))
        node_blk = lambda i: (lax.rem(i, nb), 0)
        upper_blk = lambda i: (lax.rem(i, nb) + nb, 0)
    else:
        grid = nb
        out_shape = jax.ShapeDtypeStruct((N_NODES, D), jnp.float32)
        out_spec = pl.BlockSpec((_DENSE_BLK, D), lambda i: (i, 0))
        node_blk = lambda i: (i, 0)
        upper_blk = lambda i: (i + nb, 0)
    return pl.pallas_call(
        functools.partial(_dense_body, split_out),
        grid=(grid,),
        in_specs=[
            pl.BlockSpec((_DENSE_BLK, DH), node_blk),
            pl.BlockSpec((_DENSE_BLK, DH), node_blk),
            pl.BlockSpec((_DENSE_BLK, DH), node_blk),
            pl.BlockSpec((_DENSE_BLK, DH), upper_blk),
            pl.BlockSpec((D, D), lambda i: (0, 0)),
            pl.BlockSpec((D,), lambda i: (0,)),
            pl.BlockSpec((D, D), lambda i: (0, 0)),
        ],
        out_specs=out_spec,
        out_shape=out_shape,
    )(aggl, aggr, x2, x2, Wr, br, Wo)


_HEAD_BLK = 2048


def _head_body(oh_ref, oa_ref, w1h_ref, w1a_ref, b1_ref,
               w2_ref, b2_ref, w3_ref, b3_ref, w4_ref, b4_ref,
               w5_ref, b5_ref, o_ref):
    def mm(a, w):
        return lax.dot_general(a, w, (((1,), (1,)), ((), ())),
                               precision=lax.Precision.HIGHEST,
                               preferred_element_type=jnp.float32)

    h = mm(oh_ref[...], w1h_ref[...]) + mm(oa_ref[...], w1a_ref[...])
    h = _leaky(h + b1_ref[...][None, :])
    for w_ref, b_ref in ((w2_ref, b2_ref), (w3_ref, b3_ref), (w4_ref, b4_ref)):
        h = _leaky(mm(h, w_ref[...]) + b_ref[...][None, :])
    h = _leaky(mm(h, w5_ref[...]) + b5_ref[...][None, :])
    m = jnp.max(h, axis=1, keepdims=True)
    sh = h - m
    lse = jnp.log(jnp.sum(jnp.exp(sh), axis=1, keepdims=True))
    o_ref[...] = sh - lse


def _head(oh, oa, w1h, w1a, b1, w2, b2, w3, b3, w4, b4, w5, b5):
    grid = B_PAIRS // _HEAD_BLK
    full = lambda shape: pl.BlockSpec(shape, lambda i: tuple(0 for _ in shape))
    return pl.pallas_call(
        _head_body,
        grid=(grid,),
        in_specs=[
            pl.BlockSpec((_HEAD_BLK, D), lambda i: (i, 0)),
            pl.BlockSpec((_HEAD_BLK, D), lambda i: (i, 0)),
            full((8, D)), full((8, D)), full((8,)),
            full((8, 8)), full((8,)),
            full((8, 8)), full((8,)),
            full((8, 8)), full((8,)),
            full((3, 8)), full((3,)),
        ],
        out_specs=pl.BlockSpec((_HEAD_BLK, 3), lambda i: (i, 0)),
        out_shape=jax.ShapeDtypeStruct((B_PAIRS, 3), jnp.float32),
    )(oh, oa, w1h, w1a, b1, w2, b2, w3, b3, w4, b4, w5, b5)


def kernel(edge_index, edge_weight, home, away, embed,
           W_rel1, b_rel1, W_root1, W_rel2, b_rel2, W_root2,
           W_rel3, b_rel3, W_root3,
           Wd1, bd1, Wd2, bd2, Wd3, bd3, Wd4, bd4, Wd5, bd5):
    pad = E_PAD - E_EDGES
    src2 = jnp.pad(edge_index[0], (0, pad)).reshape(-1, 128)
    dst2 = jnp.pad(edge_index[1], (0, pad)).reshape(-1, 128)
    w2 = jnp.pad(edge_weight, (0, pad)).reshape(-1, 128)
    home2 = home.reshape(-1, 128)
    away2 = away.reshape(-1, 128)

    x2 = jnp.concatenate([embed[:, :DH], embed[:, DH:]], axis=0)
    for li, (Wr, br, Wo) in enumerate(((W_rel1, b_rel1, W_root1),
                                       (W_rel2, b_rel2, W_root2),
                                       (W_rel3, b_rel3, W_root3))):
        aggl, aggr = _segsum(src2, dst2, w2, x2)
        if li < 2:
            x2 = _dense(aggl, aggr, x2, Wr, br, Wo, split_out=True)
        else:
            x = _dense(aggl, aggr, x2, Wr, br, Wo, split_out=False)

    oh, oa = _gather_pair(x, home2, away2)
    return _head(oh, oa, Wd1[:, :DH * 2], Wd1[:, DH * 2:], bd1,
                 Wd2, bd2, Wd3, bd3, Wd4, bd4, Wd5, bd5)
